# baseline (speedup 1.0000x reference)
import functools

import jax
import jax.numpy as jnp
from jax import lax
from jax.experimental import pallas as pl
from jax.experimental.pallas import tpu as pltpu
from jax.experimental.pallas import tpu_sc as plsc

N = 10000
E = 320000
E2 = 327680
NSLICE = 8
ES = E2 // NSLICE
H = 128
NC = 2
NS = 16
NW = NC * NS
GCH = 64
SCH = 2048
F32 = jnp.float32

_mesh = plsc.VectorSubcoreMesh(core_axis_name="c", subcore_axis_name="s")
_sc_params = pltpu.CompilerParams(needs_layout_passes=False)


def _worker_id():
    return lax.axis_index("s") * NC + lax.axis_index("c")


def _precompute_body(h_ref, w1r_ref, w1c_ref, b1_ref, a_ref, b_ref):
    a_ref[...] = jnp.dot(h_ref[...], w1r_ref[...], preferred_element_type=F32)
    b_ref[...] = (jnp.dot(h_ref[...], w1c_ref[...], preferred_element_type=F32)
                  + b1_ref[...])


def _precompute(h, w1r, w1c, b1):
    return pl.pallas_call(
        _precompute_body,
        out_shape=[jax.ShapeDtypeStruct((N, H), F32),
                   jax.ShapeDtypeStruct((N, H), F32)],
    )(h, w1r, w1c, b1)


def _make_gather(ne):
    epw = ne // NW

    def _gather_body(ap_hbm, bp_hbm, row_hbm, col_hbm, x0_hbm, x1_hbm, x2_hbm,
                     ga_hbm, d0_hbm, d1_hbm, d2_hbm, rad_hbm,
                     idxr, idxc, ba0, ba1, bb0, bb1, x0v, x1v, x2v,
                     d0v, d1v, d2v, radv, sga0, sgb0, sga1, sgb1, swr):
        wid = _worker_id()
        base0 = wid * epw
        pltpu.sync_copy(x0_hbm, x0v)
        pltpu.sync_copy(x1_hbm, x1v)
        pltpu.sync_copy(x2_hbm, x2v)
        pltpu.sync_copy(row_hbm.at[pl.ds(base0, epw)], idxr)
        pltpu.sync_copy(col_hbm.at[pl.ds(base0, epw)], idxc)

        def coords(j, _):
            sl = pl.ds(j * 16, 16)
            r = idxr[sl]
            c = idxc[sl]
            d0 = plsc.load_gather(x0v, [r]) - plsc.load_gather(x0v, [c])
            d1 = plsc.load_gather(x1v, [r]) - plsc.load_gather(x1v, [c])
            d2 = plsc.load_gather(x2v, [r]) - plsc.load_gather(x2v, [c])
            d0v[sl] = d0
            d1v[sl] = d1
            d2v[sl] = d2
            radv[sl] = d0 * d0 + d1 * d1 + d2 * d2
            return 0

        lax.fori_loop(0, epw // 16, coords, 0)
        pltpu.sync_copy(d0v, d0_hbm.at[pl.ds(base0, epw)])
        pltpu.sync_copy(d1v, d1_hbm.at[pl.ds(base0, epw)])
        pltpu.sync_copy(d2v, d2_hbm.at[pl.ds(base0, epw)])
        pltpu.sync_copy(radv, rad_hbm.at[pl.ds(base0, epw)])

        def _accum(dst, src):
            def addrow(i, _):
                for k in range(H // 16):
                    sl = pl.ds(k * 16, 16)
                    dst[i, sl] = dst[i, sl] + src[i, sl]
                return 0

            lax.fori_loop(0, GCH, addrow, 0)

        def gpair(p, _):
            off0 = p * 2 * GCH
            off1 = off0 + GCH
            ca0 = pltpu.async_copy(ap_hbm.at[idxr.at[pl.ds(off0, GCH)]], ba0, sga0)
            cb0 = pltpu.async_copy(bp_hbm.at[idxc.at[pl.ds(off0, GCH)]], bb0, sgb0)
            ca1 = pltpu.async_copy(ap_hbm.at[idxr.at[pl.ds(off1, GCH)]], ba1, sga1)
            cb1 = pltpu.async_copy(bp_hbm.at[idxc.at[pl.ds(off1, GCH)]], bb1, sgb1)
            ca0.wait()
            cb0.wait()
            _accum(ba0, bb0)
            w0 = pltpu.async_copy(ba0, ga_hbm.at[pl.ds(base0 + off0, GCH)], swr)
            ca1.wait()
            cb1.wait()
            _accum(ba1, bb1)
            w1 = pltpu.async_copy(ba1, ga_hbm.at[pl.ds(base0 + off1, GCH)], swr)
            w0.wait()
            w1.wait()
            return 0

        lax.fori_loop(0, epw // (2 * GCH), gpair, 0)

    return functools.partial(
        pl.kernel,
        out_type=[jax.ShapeDtypeStruct((ne, H), F32),
                  jax.ShapeDtypeStruct((ne,), F32),
                  jax.ShapeDtypeStruct((ne,), F32),
                  jax.ShapeDtypeStruct((ne,), F32),
                  jax.ShapeDtypeStruct((ne,), F32)],
        mesh=_mesh,
        scratch_types=[
            pltpu.VMEM((epw,), jnp.int32),
            pltpu.VMEM((epw,), jnp.int32),
            pltpu.VMEM((GCH, H), F32),
            pltpu.VMEM((GCH, H), F32),
            pltpu.VMEM((GCH, H), F32),
            pltpu.VMEM((GCH, H), F32),
            pltpu.VMEM((N,), F32),
            pltpu.VMEM((N,), F32),
            pltpu.VMEM((N,), F32),
            pltpu.VMEM((epw,), F32),
            pltpu.VMEM((epw,), F32),
            pltpu.VMEM((epw,), F32),
            pltpu.VMEM((epw,), F32),
            pltpu.SemaphoreType.DMA,
            pltpu.SemaphoreType.DMA,
            pltpu.SemaphoreType.DMA,
            pltpu.SemaphoreType.DMA,
            pltpu.SemaphoreType.DMA,
        ],
        compiler_params=_sc_params,
    )(_gather_body)


_gather_slice = _make_gather(ES)


EB = 2048


def _mlp_body(g_ref, eat_ref, d0_ref, d1_ref, d2_ref, rad_ref,
              w1rad_ref, w2_ref, b2_ref, w3_ref,
              t0_ref, t1_ref, t2_ref):
    rad1 = rad_ref[...]
    radial = rad1.reshape(EB, 1)
    ea_term = lax.dot_general(eat_ref[...], w1rad_ref[...],
                              (((0,), (0,)), ((), ())),
                              preferred_element_type=F32)
    u = (g_ref[...] + ea_term
         + radial * w1rad_ref[7:8, :])
    t = u * jax.nn.sigmoid(u)
    v = jnp.dot(t, w2_ref[...], preferred_element_type=F32) + b2_ref[...]
    t = v * jax.nn.sigmoid(v)
    s = lax.dot_general(w3_ref[...], t, (((1,), (1,)), ((), ())),
                        preferred_element_type=F32).reshape(EB)
    f = s / (jnp.sqrt(rad1 + 1e-8) + 1.0)
    t0_ref[...] = d0_ref[...] * f
    t1_ref[...] = d1_ref[...] * f
    t2_ref[...] = d2_ref[...] * f


def _mlp(g, eat, d0, d1, d2, rad, w1tail, w2, b2, w3):
    ne = g.shape[0]
    grid = (ne // EB,)
    vec_spec = pl.BlockSpec((EB,), lambda i: (i,))
    return pl.pallas_call(
        _mlp_body,
        grid=grid,
        in_specs=[
            pl.BlockSpec((EB, H), lambda i: (i, 0)),
            pl.BlockSpec((8, EB), lambda i: (0, i)),
            vec_spec, vec_spec, vec_spec, vec_spec,
            pl.BlockSpec((8, H), lambda i: (0, 0)),
            pl.BlockSpec((H, H), lambda i: (0, 0)),
            pl.BlockSpec((1, H), lambda i: (0, 0)),
            pl.BlockSpec((1, H), lambda i: (0, 0)),
        ],
        out_specs=[vec_spec, vec_spec, vec_spec],
        out_shape=[jax.ShapeDtypeStruct((ne,), F32)] * 3,
    )(g, eat, d0, d1, d2, rad, w1tail, w2, b2, w3)


def _scatter_body(row_hbm, t0_hbm, t1_hbm, t2_hbm, parts_hbm,
                  rowv, v0, v1, v2, acc0, acc1, acc2):
    wid = _worker_id()
    epw = E2 // NW
    base0 = wid * epw

    def zero(i, _):
        sl = pl.ds(i * 16, 16)
        z = jnp.zeros((16,), F32)
        acc0[sl] = z
        acc1[sl] = z
        acc2[sl] = z
        return 0

    lax.fori_loop(0, N // 16, zero, 0)

    def outer(ci, _):
        base = base0 + ci * SCH
        pltpu.sync_copy(row_hbm.at[pl.ds(base, SCH)], rowv)
        pltpu.sync_copy(t0_hbm.at[pl.ds(base, SCH)], v0)
        pltpu.sync_copy(t1_hbm.at[pl.ds(base, SCH)], v1)
        pltpu.sync_copy(t2_hbm.at[pl.ds(base, SCH)], v2)

        def inner(j, _):
            sl = pl.ds(j * 16, 16)
            r = rowv[sl]
            plsc.addupdate_scatter(acc0, [r], v0[sl])
            plsc.addupdate_scatter(acc1, [r], v1[sl])
            plsc.addupdate_scatter(acc2, [r], v2[sl])
            return 0

        lax.fori_loop(0, SCH // 16, inner, 0)
        return 0

    lax.fori_loop(0, (E2 // NW) // SCH, outer, 0)

    pbase = wid * (3 * N)
    pltpu.sync_copy(acc0, parts_hbm.at[pl.ds(pbase, N)])
    pltpu.sync_copy(acc1, parts_hbm.at[pl.ds(pbase + N, N)])
    pltpu.sync_copy(acc2, parts_hbm.at[pl.ds(pbase + 2 * N, N)])


_scatter = functools.partial(
    pl.kernel,
    out_type=jax.ShapeDtypeStruct((NW * 3 * N,), F32),
    mesh=_mesh,
    scratch_types=[
        pltpu.VMEM((SCH,), jnp.int32),
        pltpu.VMEM((SCH,), F32),
        pltpu.VMEM((SCH,), F32),
        pltpu.VMEM((SCH,), F32),
        pltpu.VMEM((N,), F32),
        pltpu.VMEM((N,), F32),
        pltpu.VMEM((N,), F32),
    ],
    compiler_params=_sc_params,
)(_scatter_body)


def _combine_body(parts_ref, xt_ref, out_ref):
    s = jnp.sum(parts_ref[...], axis=0)
    out_ref[...] = xt_ref[...] + s * 0.01


def _combine(parts, xt):
    return pl.pallas_call(
        _combine_body,
        out_shape=jax.ShapeDtypeStruct((3, N), F32),
    )(parts, xt)


def kernel(h, x, edge_index, edge_attr, W1, b1, W2, b2, W3):
    pad = E2 - E
    padidx = jnp.arange(pad, dtype=jnp.int32) % N
    row = jnp.concatenate([edge_index[0].astype(jnp.int32), padidx])
    col = jnp.concatenate([edge_index[1].astype(jnp.int32), padidx])
    eat = jnp.pad(edge_attr.T, ((0, 1), (0, pad)))
    xt = x.T
    w1r = W1[:H]
    w1c = W1[H:2 * H]
    w1tail = W1[2 * H:]
    b2r = b2.reshape(1, H)
    w3r = W3.reshape(1, H)
    ap, bp = _precompute(h, w1r, w1c, b1.reshape(1, H))
    t0s, t1s, t2s = [], [], []
    for s in range(NSLICE):
        lo = s * ES
        g, d0, d1, d2, rad = _gather_slice(
            ap, bp, row[lo:lo + ES], col[lo:lo + ES], xt[0], xt[1], xt[2])
        t0, t1, t2 = _mlp(g, eat[:, lo:lo + ES], d0, d1, d2, rad,
                          w1tail, W2, b2r, w3r)
        t0s.append(t0)
        t1s.append(t1)
        t2s.append(t2)
    parts = _scatter(row, jnp.concatenate(t0s), jnp.concatenate(t1s),
                     jnp.concatenate(t2s))
    xnt = _combine(parts.reshape(NW, 3, N), xt)
    return (h, xnt.T)


# speedup vs baseline: 1.1053x; 1.1053x over previous
import functools

import jax
import jax.numpy as jnp
from jax import lax
from jax.experimental import pallas as pl
from jax.experimental.pallas import tpu as pltpu
from jax.experimental.pallas import tpu_sc as plsc

N = 10000
E = 320000
E2 = 327680
NSLICE = 4
ES = E2 // NSLICE
H = 128
NC = 2
NS = 16
NW = NC * NS
GCH = 64
SCH = 2048
F32 = jnp.float32

_mesh = plsc.VectorSubcoreMesh(core_axis_name="c", subcore_axis_name="s")
_sc_params = pltpu.CompilerParams(needs_layout_passes=False)


def _worker_id():
    return lax.axis_index("s") * NC + lax.axis_index("c")


def _precompute_body(h_ref, w1r_ref, w1c_ref, b1_ref, a_ref, b_ref):
    a_ref[...] = jnp.dot(h_ref[...], w1r_ref[...], preferred_element_type=F32)
    b_ref[...] = (jnp.dot(h_ref[...], w1c_ref[...], preferred_element_type=F32)
                  + b1_ref[...])


def _precompute(h, w1r, w1c, b1):
    return pl.pallas_call(
        _precompute_body,
        out_shape=[jax.ShapeDtypeStruct((N, H), F32),
                   jax.ShapeDtypeStruct((N, H), F32)],
    )(h, w1r, w1c, b1)


def _make_gather(ne):
    epw = ne // NW

    def _gather_body(ap_hbm, bp_hbm, row_hbm, col_hbm, x0_hbm, x1_hbm, x2_hbm,
                     ga_hbm, rad_hbm,
                     idxr, idxc, ba0, ba1, bb0, bb1, x0v, x1v, x2v,
                     radv, sga0, sgb0, sga1, sgb1, swr):
        wid = _worker_id()
        base0 = wid * epw
        pltpu.sync_copy(x0_hbm, x0v)
        pltpu.sync_copy(x1_hbm, x1v)
        pltpu.sync_copy(x2_hbm, x2v)
        pltpu.sync_copy(row_hbm.at[pl.ds(base0, epw)], idxr)
        pltpu.sync_copy(col_hbm.at[pl.ds(base0, epw)], idxc)

        def coords(j, _):
            sl = pl.ds(j * 16, 16)
            r = idxr[sl]
            c = idxc[sl]
            d0 = plsc.load_gather(x0v, [r]) - plsc.load_gather(x0v, [c])
            d1 = plsc.load_gather(x1v, [r]) - plsc.load_gather(x1v, [c])
            d2 = plsc.load_gather(x2v, [r]) - plsc.load_gather(x2v, [c])
            radv[sl] = d0 * d0 + d1 * d1 + d2 * d2
            return 0

        lax.fori_loop(0, epw // 16, coords, 0)
        pltpu.sync_copy(radv, rad_hbm.at[pl.ds(base0, epw)])

        def _accum(dst, src):
            def addrow(i, _):
                for k in range(H // 16):
                    sl = pl.ds(k * 16, 16)
                    dst[i, sl] = dst[i, sl] + src[i, sl]
                return 0

            lax.fori_loop(0, GCH, addrow, 0)

        def gpair(p, _):
            off0 = p * 2 * GCH
            off1 = off0 + GCH
            ca0 = pltpu.async_copy(ap_hbm.at[idxr.at[pl.ds(off0, GCH)]], ba0, sga0)
            cb0 = pltpu.async_copy(bp_hbm.at[idxc.at[pl.ds(off0, GCH)]], bb0, sgb0)
            ca1 = pltpu.async_copy(ap_hbm.at[idxr.at[pl.ds(off1, GCH)]], ba1, sga1)
            cb1 = pltpu.async_copy(bp_hbm.at[idxc.at[pl.ds(off1, GCH)]], bb1, sgb1)
            ca0.wait()
            cb0.wait()
            _accum(ba0, bb0)
            w0 = pltpu.async_copy(ba0, ga_hbm.at[pl.ds(base0 + off0, GCH)], swr)
            ca1.wait()
            cb1.wait()
            _accum(ba1, bb1)
            w1 = pltpu.async_copy(ba1, ga_hbm.at[pl.ds(base0 + off1, GCH)], swr)
            w0.wait()
            w1.wait()
            return 0

        lax.fori_loop(0, epw // (2 * GCH), gpair, 0)

    return functools.partial(
        pl.kernel,
        out_type=[jax.ShapeDtypeStruct((ne, H), F32),
                  jax.ShapeDtypeStruct((ne,), F32)],
        mesh=_mesh,
        scratch_types=[
            pltpu.VMEM((epw,), jnp.int32),
            pltpu.VMEM((epw,), jnp.int32),
            pltpu.VMEM((GCH, H), F32),
            pltpu.VMEM((GCH, H), F32),
            pltpu.VMEM((GCH, H), F32),
            pltpu.VMEM((GCH, H), F32),
            pltpu.VMEM((N,), F32),
            pltpu.VMEM((N,), F32),
            pltpu.VMEM((N,), F32),
            pltpu.VMEM((epw,), F32),
            pltpu.SemaphoreType.DMA,
            pltpu.SemaphoreType.DMA,
            pltpu.SemaphoreType.DMA,
            pltpu.SemaphoreType.DMA,
            pltpu.SemaphoreType.DMA,
        ],
        compiler_params=_sc_params,
    )(_gather_body)


_gather_slice = _make_gather(ES)


EB = 2048


def _mlp_body(g_ref, eat_ref, rad_ref,
              w1rad_ref, w2_ref, b2_ref, w3_ref, f_ref):
    rad1 = rad_ref[...]
    radial = rad1.reshape(EB, 1)
    ea_term = lax.dot_general(eat_ref[...], w1rad_ref[...],
                              (((0,), (0,)), ((), ())),
                              preferred_element_type=F32)
    u = (g_ref[...] + ea_term
         + radial * w1rad_ref[7:8, :])
    t = u * jax.nn.sigmoid(u)
    v = jnp.dot(t, w2_ref[...], preferred_element_type=F32) + b2_ref[...]
    t = v * jax.nn.sigmoid(v)
    s = lax.dot_general(w3_ref[...], t, (((1,), (1,)), ((), ())),
                        preferred_element_type=F32).reshape(EB)
    f_ref[...] = s / (jnp.sqrt(rad1 + 1e-8) + 1.0)


def _mlp(g, eat, rad, w1tail, w2, b2, w3):
    ne = g.shape[0]
    grid = (ne // EB,)
    vec_spec = pl.BlockSpec((EB,), lambda i: (i,))
    return pl.pallas_call(
        _mlp_body,
        grid=grid,
        in_specs=[
            pl.BlockSpec((EB, H), lambda i: (i, 0)),
            pl.BlockSpec((8, EB), lambda i: (0, i)),
            vec_spec,
            pl.BlockSpec((8, H), lambda i: (0, 0)),
            pl.BlockSpec((H, H), lambda i: (0, 0)),
            pl.BlockSpec((1, H), lambda i: (0, 0)),
            pl.BlockSpec((1, H), lambda i: (0, 0)),
        ],
        out_specs=vec_spec,
        out_shape=jax.ShapeDtypeStruct((ne,), F32),
    )(g, eat, rad, w1tail, w2, b2, w3)


def _scatter_body(row_hbm, col_hbm, f_hbm, x0_hbm, x1_hbm, x2_hbm, parts_hbm,
                  rowv, colv, fv, x0v, x1v, x2v, acc0, acc1, acc2):
    wid = _worker_id()
    epw = E2 // NW
    base0 = wid * epw
    pltpu.sync_copy(x0_hbm, x0v)
    pltpu.sync_copy(x1_hbm, x1v)
    pltpu.sync_copy(x2_hbm, x2v)

    def zero(i, _):
        sl = pl.ds(i * 16, 16)
        z = jnp.zeros((16,), F32)
        acc0[sl] = z
        acc1[sl] = z
        acc2[sl] = z
        return 0

    lax.fori_loop(0, N // 16, zero, 0)

    def outer(ci, _):
        base = base0 + ci * SCH
        pltpu.sync_copy(row_hbm.at[pl.ds(base, SCH)], rowv)
        pltpu.sync_copy(col_hbm.at[pl.ds(base, SCH)], colv)
        pltpu.sync_copy(f_hbm.at[pl.ds(base, SCH)], fv)

        def inner(j, _):
            sl = pl.ds(j * 16, 16)
            r = rowv[sl]
            c = colv[sl]
            f = fv[sl]
            d0 = plsc.load_gather(x0v, [r]) - plsc.load_gather(x0v, [c])
            d1 = plsc.load_gather(x1v, [r]) - plsc.load_gather(x1v, [c])
            d2 = plsc.load_gather(x2v, [r]) - plsc.load_gather(x2v, [c])
            plsc.addupdate_scatter(acc0, [r], d0 * f)
            plsc.addupdate_scatter(acc1, [r], d1 * f)
            plsc.addupdate_scatter(acc2, [r], d2 * f)
            return 0

        lax.fori_loop(0, SCH // 16, inner, 0)
        return 0

    lax.fori_loop(0, (E2 // NW) // SCH, outer, 0)

    pbase = wid * (3 * N)
    pltpu.sync_copy(acc0, parts_hbm.at[pl.ds(pbase, N)])
    pltpu.sync_copy(acc1, parts_hbm.at[pl.ds(pbase + N, N)])
    pltpu.sync_copy(acc2, parts_hbm.at[pl.ds(pbase + 2 * N, N)])


_scatter = functools.partial(
    pl.kernel,
    out_type=jax.ShapeDtypeStruct((NW * 3 * N,), F32),
    mesh=_mesh,
    scratch_types=[
        pltpu.VMEM((SCH,), jnp.int32),
        pltpu.VMEM((SCH,), jnp.int32),
        pltpu.VMEM((SCH,), F32),
        pltpu.VMEM((N,), F32),
        pltpu.VMEM((N,), F32),
        pltpu.VMEM((N,), F32),
        pltpu.VMEM((N,), F32),
        pltpu.VMEM((N,), F32),
        pltpu.VMEM((N,), F32),
    ],
    compiler_params=_sc_params,
)(_scatter_body)


def _combine_body(parts_ref, xt_ref, out_ref):
    s = jnp.sum(parts_ref[...], axis=0)
    out_ref[...] = xt_ref[...] + s * 0.01


def _combine(parts, xt):
    return pl.pallas_call(
        _combine_body,
        out_shape=jax.ShapeDtypeStruct((3, N), F32),
    )(parts, xt)


def kernel(h, x, edge_index, edge_attr, W1, b1, W2, b2, W3):
    pad = E2 - E
    padidx = jnp.arange(pad, dtype=jnp.int32) % N
    row = jnp.concatenate([edge_index[0].astype(jnp.int32), padidx])
    col = jnp.concatenate([edge_index[1].astype(jnp.int32), padidx])
    eat = jnp.pad(edge_attr.T, ((0, 1), (0, pad)))
    xt = x.T
    w1r = W1[:H]
    w1c = W1[H:2 * H]
    w1tail = W1[2 * H:]
    b2r = b2.reshape(1, H)
    w3r = W3.reshape(1, H)
    ap, bp = _precompute(h, w1r, w1c, b1.reshape(1, H))
    fs = []
    for s in range(NSLICE):
        lo = s * ES
        g, rad = _gather_slice(
            ap, bp, row[lo:lo + ES], col[lo:lo + ES], xt[0], xt[1], xt[2])
        fs.append(_mlp(g, eat[:, lo:lo + ES], rad, w1tail, W2, b2r, w3r))
    parts = _scatter(row, col, jnp.concatenate(fs), xt[0], xt[1], xt[2])
    xnt = _combine(parts.reshape(NW, 3, N), xt)
    return (h, xnt.T)


# speedup vs baseline: 1.1532x; 1.0433x over previous
import functools

import jax
import jax.numpy as jnp
from jax import lax
from jax.experimental import pallas as pl
from jax.experimental.pallas import tpu as pltpu
from jax.experimental.pallas import tpu_sc as plsc

N = 10000
E = 320000
E2 = 327680
NSLICE = 4
ES = E2 // NSLICE
H = 128
NC = 2
NS = 16
NW = NC * NS
GCH = 128
SCH = 2048
F32 = jnp.float32

_mesh = plsc.VectorSubcoreMesh(core_axis_name="c", subcore_axis_name="s")
_sc_params = pltpu.CompilerParams(needs_layout_passes=False)


def _worker_id():
    return lax.axis_index("s") * NC + lax.axis_index("c")


def _precompute_body(h_ref, w1r_ref, w1c_ref, b1_ref, a_ref, b_ref):
    a_ref[...] = jnp.dot(h_ref[...], w1r_ref[...], preferred_element_type=F32)
    b_ref[...] = (jnp.dot(h_ref[...], w1c_ref[...], preferred_element_type=F32)
                  + b1_ref[...])


def _precompute(h, w1r, w1c, b1):
    return pl.pallas_call(
        _precompute_body,
        out_shape=[jax.ShapeDtypeStruct((N, H), F32),
                   jax.ShapeDtypeStruct((N, H), F32)],
    )(h, w1r, w1c, b1)


def _make_gather(ne):
    epw = ne // NW

    def _gather_body(ap_hbm, bp_hbm, row_hbm, col_hbm, x0_hbm, x1_hbm, x2_hbm,
                     ga_hbm, rad_hbm,
                     idxr, idxc, ba0, ba1, bb0, bb1, x0v, x1v, x2v,
                     radv, sga0, sgb0, sga1, sgb1, swr):
        wid = _worker_id()
        base0 = wid * epw
        pltpu.sync_copy(x0_hbm, x0v)
        pltpu.sync_copy(x1_hbm, x1v)
        pltpu.sync_copy(x2_hbm, x2v)
        pltpu.sync_copy(row_hbm.at[pl.ds(base0, epw)], idxr)
        pltpu.sync_copy(col_hbm.at[pl.ds(base0, epw)], idxc)

        def coords(j, _):
            sl = pl.ds(j * 16, 16)
            r = idxr[sl]
            c = idxc[sl]
            d0 = plsc.load_gather(x0v, [r]) - plsc.load_gather(x0v, [c])
            d1 = plsc.load_gather(x1v, [r]) - plsc.load_gather(x1v, [c])
            d2 = plsc.load_gather(x2v, [r]) - plsc.load_gather(x2v, [c])
            radv[sl] = d0 * d0 + d1 * d1 + d2 * d2
            return 0

        lax.fori_loop(0, epw // 16, coords, 0)
        pltpu.sync_copy(radv, rad_hbm.at[pl.ds(base0, epw)])

        def _accum(dst, src):
            def addrow(i, _):
                for k in range(H // 16):
                    sl = pl.ds(k * 16, 16)
                    dst[i, sl] = dst[i, sl] + src[i, sl]
                return 0

            lax.fori_loop(0, GCH, addrow, 0)

        def gpair(p, _):
            off0 = p * 2 * GCH
            off1 = off0 + GCH
            ca0 = pltpu.async_copy(ap_hbm.at[idxr.at[pl.ds(off0, GCH)]], ba0, sga0)
            cb0 = pltpu.async_copy(bp_hbm.at[idxc.at[pl.ds(off0, GCH)]], bb0, sgb0)
            ca1 = pltpu.async_copy(ap_hbm.at[idxr.at[pl.ds(off1, GCH)]], ba1, sga1)
            cb1 = pltpu.async_copy(bp_hbm.at[idxc.at[pl.ds(off1, GCH)]], bb1, sgb1)
            ca0.wait()
            cb0.wait()
            _accum(ba0, bb0)
            w0 = pltpu.async_copy(ba0, ga_hbm.at[pl.ds(base0 + off0, GCH)], swr)
            ca1.wait()
            cb1.wait()
            _accum(ba1, bb1)
            w1 = pltpu.async_copy(ba1, ga_hbm.at[pl.ds(base0 + off1, GCH)], swr)
            w0.wait()
            w1.wait()
            return 0

        lax.fori_loop(0, epw // (2 * GCH), gpair, 0)

    return functools.partial(
        pl.kernel,
        out_type=[jax.ShapeDtypeStruct((ne, H), F32),
                  jax.ShapeDtypeStruct((ne,), F32)],
        mesh=_mesh,
        scratch_types=[
            pltpu.VMEM((epw,), jnp.int32),
            pltpu.VMEM((epw,), jnp.int32),
            pltpu.VMEM((GCH, H), F32),
            pltpu.VMEM((GCH, H), F32),
            pltpu.VMEM((GCH, H), F32),
            pltpu.VMEM((GCH, H), F32),
            pltpu.VMEM((N,), F32),
            pltpu.VMEM((N,), F32),
            pltpu.VMEM((N,), F32),
            pltpu.VMEM((epw,), F32),
            pltpu.SemaphoreType.DMA,
            pltpu.SemaphoreType.DMA,
            pltpu.SemaphoreType.DMA,
            pltpu.SemaphoreType.DMA,
            pltpu.SemaphoreType.DMA,
        ],
        compiler_params=_sc_params,
    )(_gather_body)


_gather_slice = _make_gather(ES)


EB = 2048


def _mlp_body(g_ref, eat_ref, rad_ref,
              w1rad_ref, w2_ref, b2_ref, w3_ref, f_ref):
    rad1 = rad_ref[...]
    radial = rad1.reshape(EB, 1)
    ea_term = lax.dot_general(eat_ref[...], w1rad_ref[...],
                              (((0,), (0,)), ((), ())),
                              preferred_element_type=F32)
    u = (g_ref[...] + ea_term
         + radial * w1rad_ref[7:8, :])
    t = u * jax.nn.sigmoid(u)
    v = jnp.dot(t, w2_ref[...], preferred_element_type=F32) + b2_ref[...]
    t = v * jax.nn.sigmoid(v)
    s = lax.dot_general(w3_ref[...], t, (((1,), (1,)), ((), ())),
                        preferred_element_type=F32).reshape(EB)
    f_ref[...] = s / (jnp.sqrt(rad1 + 1e-8) + 1.0)


def _mlp(g, eat, rad, w1tail, w2, b2, w3):
    ne = g.shape[0]
    grid = (ne // EB,)
    vec_spec = pl.BlockSpec((EB,), lambda i: (i,))
    return pl.pallas_call(
        _mlp_body,
        grid=grid,
        in_specs=[
            pl.BlockSpec((EB, H), lambda i: (i, 0)),
            pl.BlockSpec((8, EB), lambda i: (0, i)),
            vec_spec,
            pl.BlockSpec((8, H), lambda i: (0, 0)),
            pl.BlockSpec((H, H), lambda i: (0, 0)),
            pl.BlockSpec((1, H), lambda i: (0, 0)),
            pl.BlockSpec((1, H), lambda i: (0, 0)),
        ],
        out_specs=vec_spec,
        out_shape=jax.ShapeDtypeStruct((ne,), F32),
    )(g, eat, rad, w1tail, w2, b2, w3)


def _scatter_body(row_hbm, col_hbm, f_hbm, x0_hbm, x1_hbm, x2_hbm, parts_hbm,
                  rowv, colv, fv, x0v, x1v, x2v, acc0, acc1, acc2):
    wid = _worker_id()
    epw = E2 // NW
    base0 = wid * epw
    pltpu.sync_copy(x0_hbm, x0v)
    pltpu.sync_copy(x1_hbm, x1v)
    pltpu.sync_copy(x2_hbm, x2v)

    def zero(i, _):
        sl = pl.ds(i * 16, 16)
        z = jnp.zeros((16,), F32)
        acc0[sl] = z
        acc1[sl] = z
        acc2[sl] = z
        return 0

    lax.fori_loop(0, N // 16, zero, 0)

    def outer(ci, _):
        base = base0 + ci * SCH
        pltpu.sync_copy(row_hbm.at[pl.ds(base, SCH)], rowv)
        pltpu.sync_copy(col_hbm.at[pl.ds(base, SCH)], colv)
        pltpu.sync_copy(f_hbm.at[pl.ds(base, SCH)], fv)

        def inner(j, _):
            sl = pl.ds(j * 16, 16)
            r = rowv[sl]
            c = colv[sl]
            f = fv[sl]
            d0 = plsc.load_gather(x0v, [r]) - plsc.load_gather(x0v, [c])
            d1 = plsc.load_gather(x1v, [r]) - plsc.load_gather(x1v, [c])
            d2 = plsc.load_gather(x2v, [r]) - plsc.load_gather(x2v, [c])
            plsc.addupdate_scatter(acc0, [r], d0 * f)
            plsc.addupdate_scatter(acc1, [r], d1 * f)
            plsc.addupdate_scatter(acc2, [r], d2 * f)
            return 0

        lax.fori_loop(0, SCH // 16, inner, 0)
        return 0

    lax.fori_loop(0, (E2 // NW) // SCH, outer, 0)

    pbase = wid * (3 * N)
    pltpu.sync_copy(acc0, parts_hbm.at[pl.ds(pbase, N)])
    pltpu.sync_copy(acc1, parts_hbm.at[pl.ds(pbase + N, N)])
    pltpu.sync_copy(acc2, parts_hbm.at[pl.ds(pbase + 2 * N, N)])


_scatter = functools.partial(
    pl.kernel,
    out_type=jax.ShapeDtypeStruct((NW * 3 * N,), F32),
    mesh=_mesh,
    scratch_types=[
        pltpu.VMEM((SCH,), jnp.int32),
        pltpu.VMEM((SCH,), jnp.int32),
        pltpu.VMEM((SCH,), F32),
        pltpu.VMEM((N,), F32),
        pltpu.VMEM((N,), F32),
        pltpu.VMEM((N,), F32),
        pltpu.VMEM((N,), F32),
        pltpu.VMEM((N,), F32),
        pltpu.VMEM((N,), F32),
    ],
    compiler_params=_sc_params,
)(_scatter_body)


def _combine_body(parts_ref, xt_ref, out_ref):
    s = jnp.sum(parts_ref[...], axis=0)
    out_ref[...] = xt_ref[...] + s * 0.01


def _combine(parts, xt):
    return pl.pallas_call(
        _combine_body,
        out_shape=jax.ShapeDtypeStruct((3, N), F32),
    )(parts, xt)


def kernel(h, x, edge_index, edge_attr, W1, b1, W2, b2, W3):
    pad = E2 - E
    padidx = jnp.arange(pad, dtype=jnp.int32) % N
    row = jnp.concatenate([edge_index[0].astype(jnp.int32), padidx])
    col = jnp.concatenate([edge_index[1].astype(jnp.int32), padidx])
    eat = jnp.pad(edge_attr.T, ((0, 1), (0, pad)))
    xt = x.T
    w1r = W1[:H]
    w1c = W1[H:2 * H]
    w1tail = W1[2 * H:]
    b2r = b2.reshape(1, H)
    w3r = W3.reshape(1, H)
    ap, bp = _precompute(h, w1r, w1c, b1.reshape(1, H))
    fs = []
    for s in range(NSLICE):
        lo = s * ES
        g, rad = _gather_slice(
            ap, bp, row[lo:lo + ES], col[lo:lo + ES], xt[0], xt[1], xt[2])
        fs.append(_mlp(g, eat[:, lo:lo + ES], rad, w1tail, W2, b2r, w3r))
    parts = _scatter(row, col, jnp.concatenate(fs), xt[0], xt[1], xt[2])
    xnt = _combine(parts.reshape(NW, 3, N), xt)
    return (h, xnt.T)


# speedup vs baseline: 1.1699x; 1.0145x over previous
import functools

import jax
import jax.numpy as jnp
from jax import lax
from jax.experimental import pallas as pl
from jax.experimental.pallas import tpu as pltpu
from jax.experimental.pallas import tpu_sc as plsc

N = 10000
E = 320000
E2 = 327680
NSLICE = 4
ES = E2 // NSLICE
H = 128
NC = 2
NS = 16
NW = NC * NS
GCH = 128
SCH = 2048
F32 = jnp.float32

_mesh = plsc.VectorSubcoreMesh(core_axis_name="c", subcore_axis_name="s")
_sc_params = pltpu.CompilerParams(needs_layout_passes=False)


def _worker_id():
    return lax.axis_index("s") * NC + lax.axis_index("c")


def _precompute_body(h_ref, w1r_ref, w1c_ref, b1_ref, a_ref, b_ref):
    a_ref[...] = jnp.dot(h_ref[...], w1r_ref[...], preferred_element_type=F32)
    b_ref[...] = (jnp.dot(h_ref[...], w1c_ref[...], preferred_element_type=F32)
                  + b1_ref[...])


def _precompute(h, w1r, w1c, b1):
    return pl.pallas_call(
        _precompute_body,
        out_shape=[jax.ShapeDtypeStruct((N, H), F32),
                   jax.ShapeDtypeStruct((N, H), F32)],
    )(h, w1r, w1c, b1)


def _make_gather(ne):
    epw = ne // NW

    def _gather_body(ap_hbm, bp_hbm, row_hbm, col_hbm, x0_hbm, x1_hbm, x2_hbm,
                     ga_hbm, rad_hbm,
                     idxr, idxc, ba0, ba1, bb0, bb1, x0v, x1v, x2v,
                     radv, sga0, sgb0, sga1, sgb1, swr):
        wid = _worker_id()
        base0 = wid * epw
        pltpu.sync_copy(x0_hbm, x0v)
        pltpu.sync_copy(x1_hbm, x1v)
        pltpu.sync_copy(x2_hbm, x2v)
        pltpu.sync_copy(row_hbm.at[pl.ds(base0, epw)], idxr)
        pltpu.sync_copy(col_hbm.at[pl.ds(base0, epw)], idxc)

        def coords(j, _):
            sl = pl.ds(j * 16, 16)
            r = idxr[sl]
            c = idxc[sl]
            d0 = plsc.load_gather(x0v, [r]) - plsc.load_gather(x0v, [c])
            d1 = plsc.load_gather(x1v, [r]) - plsc.load_gather(x1v, [c])
            d2 = plsc.load_gather(x2v, [r]) - plsc.load_gather(x2v, [c])
            radv[sl] = d0 * d0 + d1 * d1 + d2 * d2
            return 0

        lax.fori_loop(0, epw // 16, coords, 0)
        pltpu.sync_copy(radv, rad_hbm.at[pl.ds(base0, epw)])

        def _accum(dst, src):
            def addrow(i, _):
                for k in range(H // 16):
                    sl = pl.ds(k * 16, 16)
                    dst[i, sl] = dst[i, sl] + src[i, sl]
                return 0

            lax.fori_loop(0, GCH, addrow, 0)

        def gpair(p, _):
            off0 = p * 2 * GCH
            off1 = off0 + GCH
            ca0 = pltpu.async_copy(ap_hbm.at[idxr.at[pl.ds(off0, GCH)]], ba0, sga0)
            cb0 = pltpu.async_copy(bp_hbm.at[idxc.at[pl.ds(off0, GCH)]], bb0, sgb0)
            ca1 = pltpu.async_copy(ap_hbm.at[idxr.at[pl.ds(off1, GCH)]], ba1, sga1)
            cb1 = pltpu.async_copy(bp_hbm.at[idxc.at[pl.ds(off1, GCH)]], bb1, sgb1)
            ca0.wait()
            cb0.wait()
            _accum(ba0, bb0)
            w0 = pltpu.async_copy(ba0, ga_hbm.at[pl.ds(base0 + off0, GCH)], swr)
            ca1.wait()
            cb1.wait()
            _accum(ba1, bb1)
            w1 = pltpu.async_copy(ba1, ga_hbm.at[pl.ds(base0 + off1, GCH)], swr)
            w0.wait()
            w1.wait()
            return 0

        lax.fori_loop(0, epw // (2 * GCH), gpair, 0)

    return functools.partial(
        pl.kernel,
        out_type=[jax.ShapeDtypeStruct((ne, H), F32),
                  jax.ShapeDtypeStruct((ne,), F32)],
        mesh=_mesh,
        scratch_types=[
            pltpu.VMEM((epw,), jnp.int32),
            pltpu.VMEM((epw,), jnp.int32),
            pltpu.VMEM((GCH, H), F32),
            pltpu.VMEM((GCH, H), F32),
            pltpu.VMEM((GCH, H), F32),
            pltpu.VMEM((GCH, H), F32),
            pltpu.VMEM((N,), F32),
            pltpu.VMEM((N,), F32),
            pltpu.VMEM((N,), F32),
            pltpu.VMEM((epw,), F32),
            pltpu.SemaphoreType.DMA,
            pltpu.SemaphoreType.DMA,
            pltpu.SemaphoreType.DMA,
            pltpu.SemaphoreType.DMA,
            pltpu.SemaphoreType.DMA,
        ],
        compiler_params=_sc_params,
    )(_gather_body)


_gather_slice = _make_gather(ES)


EB = 2048


def _mlp_body(g_ref, eat_ref, rad_ref,
              w1rad_ref, w2_ref, b2_ref, w3_ref, f_ref):
    rad1 = rad_ref[...]
    radial = rad1.reshape(EB, 1)
    ea_term = lax.dot_general(eat_ref[...], w1rad_ref[...],
                              (((0,), (0,)), ((), ())),
                              preferred_element_type=F32)
    u = (g_ref[...] + ea_term
         + radial * w1rad_ref[7:8, :])
    t = u * jax.nn.sigmoid(u)
    v = jnp.dot(t, w2_ref[...], preferred_element_type=F32) + b2_ref[...]
    t = v * jax.nn.sigmoid(v)
    s = lax.dot_general(w3_ref[...], t, (((1,), (1,)), ((), ())),
                        preferred_element_type=F32).reshape(EB)
    f_ref[...] = s / (jnp.sqrt(rad1 + 1e-8) + 1.0)


def _mlp(g, eat, rad, w1tail, w2, b2, w3):
    ne = g.shape[0]
    grid = (ne // EB,)
    vec_spec = pl.BlockSpec((EB,), lambda i: (i,))
    return pl.pallas_call(
        _mlp_body,
        grid=grid,
        in_specs=[
            pl.BlockSpec((EB, H), lambda i: (i, 0)),
            pl.BlockSpec((8, EB), lambda i: (0, i)),
            vec_spec,
            pl.BlockSpec((8, H), lambda i: (0, 0)),
            pl.BlockSpec((H, H), lambda i: (0, 0)),
            pl.BlockSpec((1, H), lambda i: (0, 0)),
            pl.BlockSpec((1, H), lambda i: (0, 0)),
        ],
        out_specs=vec_spec,
        out_shape=jax.ShapeDtypeStruct((ne,), F32),
    )(g, eat, rad, w1tail, w2, b2, w3)


def _scatter_body(row_hbm, col_hbm, f_hbm, x0_hbm, x1_hbm, x2_hbm, parts_hbm,
                  rowv, colv, fv, x0v, x1v, x2v, acc0, acc1, acc2):
    wid = _worker_id()
    epw = E2 // NW
    base0 = wid * epw
    pltpu.sync_copy(x0_hbm, x0v)
    pltpu.sync_copy(x1_hbm, x1v)
    pltpu.sync_copy(x2_hbm, x2v)

    def zero(i, _):
        sl = pl.ds(i * 16, 16)
        z = jnp.zeros((16,), F32)
        acc0[sl] = z
        acc1[sl] = z
        acc2[sl] = z
        return 0

    lax.fori_loop(0, N // 16, zero, 0)

    pltpu.sync_copy(row_hbm.at[pl.ds(base0, epw)], rowv)
    pltpu.sync_copy(col_hbm.at[pl.ds(base0, epw)], colv)
    pltpu.sync_copy(f_hbm.at[pl.ds(base0, epw)], fv)

    def inner(j, _):
        sl = pl.ds(j * 16, 16)
        r = rowv[sl]
        c = colv[sl]
        f = fv[sl]
        d0 = plsc.load_gather(x0v, [r]) - plsc.load_gather(x0v, [c])
        d1 = plsc.load_gather(x1v, [r]) - plsc.load_gather(x1v, [c])
        d2 = plsc.load_gather(x2v, [r]) - plsc.load_gather(x2v, [c])
        plsc.addupdate_scatter(acc0, [r], d0 * f)
        plsc.addupdate_scatter(acc1, [r], d1 * f)
        plsc.addupdate_scatter(acc2, [r], d2 * f)
        return 0

    lax.fori_loop(0, epw // 16, inner, 0)

    pbase = wid * (3 * N)
    pltpu.sync_copy(acc0, parts_hbm.at[pl.ds(pbase, N)])
    pltpu.sync_copy(acc1, parts_hbm.at[pl.ds(pbase + N, N)])
    pltpu.sync_copy(acc2, parts_hbm.at[pl.ds(pbase + 2 * N, N)])


_scatter = functools.partial(
    pl.kernel,
    out_type=jax.ShapeDtypeStruct((NW * 3 * N,), F32),
    mesh=_mesh,
    scratch_types=[
        pltpu.VMEM((E2 // NW,), jnp.int32),
        pltpu.VMEM((E2 // NW,), jnp.int32),
        pltpu.VMEM((E2 // NW,), F32),
        pltpu.VMEM((N,), F32),
        pltpu.VMEM((N,), F32),
        pltpu.VMEM((N,), F32),
        pltpu.VMEM((N,), F32),
        pltpu.VMEM((N,), F32),
        pltpu.VMEM((N,), F32),
    ],
    compiler_params=_sc_params,
)(_scatter_body)


def _combine_body(parts_ref, xt_ref, out_ref):
    s = jnp.sum(parts_ref[...], axis=0)
    out_ref[...] = xt_ref[...] + s * 0.01


def _combine(parts, xt):
    return pl.pallas_call(
        _combine_body,
        out_shape=jax.ShapeDtypeStruct((3, N), F32),
    )(parts, xt)


def kernel(h, x, edge_index, edge_attr, W1, b1, W2, b2, W3):
    pad = E2 - E
    padidx = jnp.arange(pad, dtype=jnp.int32) % N
    row = jnp.concatenate([edge_index[0].astype(jnp.int32), padidx])
    col = jnp.concatenate([edge_index[1].astype(jnp.int32), padidx])
    eat = jnp.pad(edge_attr.T, ((0, 1), (0, pad)))
    xt = x.T
    w1r = W1[:H]
    w1c = W1[H:2 * H]
    w1tail = W1[2 * H:]
    b2r = b2.reshape(1, H)
    w3r = W3.reshape(1, H)
    ap, bp = _precompute(h, w1r, w1c, b1.reshape(1, H))
    fs = []
    for s in range(NSLICE):
        lo = s * ES
        g, rad = _gather_slice(
            ap, bp, row[lo:lo + ES], col[lo:lo + ES], xt[0], xt[1], xt[2])
        fs.append(_mlp(g, eat[:, lo:lo + ES], rad, w1tail, W2, b2r, w3r))
    parts = _scatter(row, col, jnp.concatenate(fs), xt[0], xt[1], xt[2])
    xnt = _combine(parts.reshape(NW, 3, N), xt)
    return (h, xnt.T)
